# baseline (device time: 162041 ns/iter reference)
import jax
import jax.numpy as jnp
from jax import lax
from jax.experimental import pallas as pl
from jax.experimental.pallas import tpu as pltpu

WORLD = 16
N_TOK = 2048
D_IN = 512
D_OUT = 1024
E_PER_DEV = 4
CHUNK = N_TOK // WORLD
N_HOPS = WORLD - 1


def kernel(x, router_W, route_idx, expert_W):
    del router_W

    xb = x.astype(jnp.bfloat16)
    wb = expert_W.astype(jnp.bfloat16)

    def body(x_ref, idx_ref, w_ref, out_ref, send_buf, recv_buf, send_sems, recv_sems):
        my = lax.axis_index("i")
        left = lax.rem(my - 1 + WORLD, WORLD)
        right = lax.rem(my + 1, WORLD)

        barrier = pltpu.get_barrier_semaphore()
        for nbr in (left, right):
            pl.semaphore_signal(
                barrier, inc=1, device_id=(nbr,), device_id_type=pl.DeviceIdType.MESH
            )
        pl.semaphore_wait(barrier, 2)

        xv = x_ref[...]
        idx = idx_ref[...]
        acc = jnp.zeros((N_TOK, D_OUT), jnp.float32)
        for e in range(E_PER_DEV):
            gid = my * E_PER_DEV + e
            xm = jnp.where(idx == gid, xv, jnp.zeros_like(xv))
            acc = acc + lax.dot(xm, w_ref[e], preferred_element_type=jnp.float32)
        out_ref[...] = acc.astype(jnp.bfloat16)

        def rows(c):
            return pl.ds(c * CHUNK, CHUNK)

        def hop(slot, sb, c_send, c_recv, is_rs):
            send_buf[sb, :, :] = out_ref[rows(c_send), :]
            rdma = pltpu.make_async_remote_copy(
                src_ref=send_buf.at[sb],
                dst_ref=recv_buf.at[slot],
                send_sem=send_sems.at[sb],
                recv_sem=recv_sems.at[slot],
                device_id=(right,),
                device_id_type=pl.DeviceIdType.MESH,
            )
            rdma.start()
            rdma.wait()
            if is_rs:
                out_ref[rows(c_recv), :] = out_ref[rows(c_recv), :] + recv_buf[slot]
            else:
                out_ref[rows(c_recv), :] = recv_buf[slot]

        for s in range(N_HOPS):
            c_send = lax.rem(my - s + 2 * WORLD, WORLD)
            c_recv = lax.rem(my - s - 1 + 2 * WORLD, WORLD)
            hop(s, s % 2, c_send, c_recv, is_rs=True)

        for t in range(N_HOPS):
            c_send = lax.rem(my + 1 - t + 2 * WORLD, WORLD)
            c_recv = lax.rem(my - t + 2 * WORLD, WORLD)
            hop(N_HOPS + t, t % 2, c_send, c_recv, is_rs=False)

    return pl.pallas_call(
        body,
        out_shape=jax.ShapeDtypeStruct((N_TOK, D_OUT), jnp.bfloat16),
        in_specs=[
            pl.BlockSpec(memory_space=pltpu.VMEM),
            pl.BlockSpec(memory_space=pltpu.VMEM),
            pl.BlockSpec(memory_space=pltpu.VMEM),
        ],
        out_specs=pl.BlockSpec(memory_space=pltpu.VMEM),
        scratch_shapes=[
            pltpu.VMEM((2, CHUNK, D_OUT), jnp.bfloat16),
            pltpu.VMEM((2 * N_HOPS, CHUNK, D_OUT), jnp.bfloat16),
            pltpu.SemaphoreType.DMA((2,)),
            pltpu.SemaphoreType.DMA((2 * N_HOPS,)),
        ],
        compiler_params=pltpu.CompilerParams(collective_id=0),
    )(xb, route_idx, wb)


# device time: 119672 ns/iter; 1.3540x vs baseline; 1.3540x over previous
import jax
import jax.numpy as jnp
from jax import lax
from jax.experimental import pallas as pl
from jax.experimental.pallas import tpu as pltpu

WORLD = 16
N_TOK = 2048
D_IN = 512
D_OUT = 1024
E_PER_DEV = 4
CHUNK = N_TOK // WORLD
N_PEER = WORLD - 1


def kernel(x, router_W, route_idx, expert_W):
    del router_W

    xb = x.astype(jnp.bfloat16)
    wb = expert_W.astype(jnp.bfloat16)

    def body(x_ref, idx_ref, w_ref, out_ref, rs_buf, ag_buf,
             rs_send_sems, rs_recv_sems, ag_send_sems, ag_recv_sems):
        my = lax.axis_index("i")

        def peer(k):
            return lax.rem(my + k, WORLD)

        def rows(c):
            return pl.ds(c * CHUNK, CHUNK)

        barrier = pltpu.get_barrier_semaphore()
        for k in range(1, WORLD):
            pl.semaphore_signal(
                barrier, inc=1, device_id=(peer(k),),
                device_id_type=pl.DeviceIdType.MESH,
            )
        pl.semaphore_wait(barrier, N_PEER)

        xv = x_ref[...]
        idx = idx_ref[...]
        acc = jnp.zeros((N_TOK, D_OUT), jnp.float32)
        for e in range(E_PER_DEV):
            gid = my * E_PER_DEV + e
            xm = jnp.where(idx == gid, xv, jnp.zeros_like(xv))
            acc = acc + lax.dot(xm, w_ref[e], preferred_element_type=jnp.float32)
        out_ref[...] = acc.astype(jnp.bfloat16)

        rs_rdmas = []
        for k in range(1, WORLD):
            rdma = pltpu.make_async_remote_copy(
                src_ref=out_ref.at[rows(peer(k)), :],
                dst_ref=rs_buf.at[k - 1],
                send_sem=rs_send_sems.at[k - 1],
                recv_sem=rs_recv_sems.at[k - 1],
                device_id=(peer(k),),
                device_id_type=pl.DeviceIdType.MESH,
            )
            rdma.start()
            rs_rdmas.append(rdma)

        for k in range(1, WORLD):
            rs_rdmas[k - 1].wait_recv()
            out_ref[rows(my), :] = out_ref[rows(my), :] + rs_buf[k - 1]

        for k in range(1, WORLD):
            rs_rdmas[k - 1].wait_send()

        ag_rdmas = []
        for k in range(1, WORLD):
            rdma = pltpu.make_async_remote_copy(
                src_ref=out_ref.at[rows(my), :],
                dst_ref=ag_buf.at[k - 1],
                send_sem=ag_send_sems.at[k - 1],
                recv_sem=ag_recv_sems.at[k - 1],
                device_id=(peer(k),),
                device_id_type=pl.DeviceIdType.MESH,
            )
            rdma.start()
            ag_rdmas.append(rdma)

        for k in range(1, WORLD):
            ag_rdmas[k - 1].wait_recv()
            src = lax.rem(my - k + WORLD, WORLD)
            out_ref[rows(src), :] = ag_buf[k - 1]

        for k in range(1, WORLD):
            ag_rdmas[k - 1].wait_send()

    return pl.pallas_call(
        body,
        out_shape=jax.ShapeDtypeStruct((N_TOK, D_OUT), jnp.bfloat16),
        in_specs=[
            pl.BlockSpec(memory_space=pltpu.VMEM),
            pl.BlockSpec(memory_space=pltpu.VMEM),
            pl.BlockSpec(memory_space=pltpu.VMEM),
        ],
        out_specs=pl.BlockSpec(memory_space=pltpu.VMEM),
        scratch_shapes=[
            pltpu.VMEM((N_PEER, CHUNK, D_OUT), jnp.bfloat16),
            pltpu.VMEM((N_PEER, CHUNK, D_OUT), jnp.bfloat16),
            pltpu.SemaphoreType.DMA((N_PEER,)),
            pltpu.SemaphoreType.DMA((N_PEER,)),
            pltpu.SemaphoreType.DMA((N_PEER,)),
            pltpu.SemaphoreType.DMA((N_PEER,)),
        ],
        compiler_params=pltpu.CompilerParams(collective_id=0),
    )(xb, route_idx, wb)


# device time: 112360 ns/iter; 1.4422x vs baseline; 1.0651x over previous
import jax
import jax.numpy as jnp
from jax import lax
from jax.experimental import pallas as pl
from jax.experimental.pallas import tpu as pltpu

WORLD = 16
N_TOK = 2048
D_IN = 512
D_OUT = 1024
E_PER_DEV = 4
CHUNK = N_TOK // WORLD
N_PEER = WORLD - 1


def kernel(x, router_W, route_idx, expert_W):
    del router_W

    xb = x.astype(jnp.bfloat16)
    wb = expert_W.astype(jnp.bfloat16)

    def body(x_ref, idx_ref, w_ref, out_ref, rs_buf, ag_buf,
             rs_send_sems, rs_recv_sems, ag_send_sems, ag_recv_sems):
        my = lax.axis_index("i")

        def peer(k):
            return lax.rem(my + k, WORLD)

        def rows(c):
            return pl.ds(c * CHUNK, CHUNK)

        barrier = pltpu.get_barrier_semaphore()
        for k in range(1, WORLD):
            pl.semaphore_signal(
                barrier, inc=1, device_id=(peer(k),),
                device_id_type=pl.DeviceIdType.MESH,
            )
        pl.semaphore_wait(barrier, N_PEER)

        rs_rdmas = []
        for i in range(WORLD):
            c = lax.rem(my + 1 + i, WORLD)
            xv = x_ref[rows(c), :]
            idx = idx_ref[rows(c), :]
            acc = jnp.zeros((CHUNK, D_OUT), jnp.float32)
            for e in range(E_PER_DEV):
                gid = my * E_PER_DEV + e
                xm = jnp.where(idx == gid, xv, jnp.zeros_like(xv))
                acc = acc + lax.dot(xm, w_ref[e], preferred_element_type=jnp.float32)
            out_ref[rows(c), :] = acc.astype(jnp.bfloat16)
            if i < N_PEER:
                rdma = pltpu.make_async_remote_copy(
                    src_ref=out_ref.at[rows(c), :],
                    dst_ref=rs_buf.at[i],
                    send_sem=rs_send_sems.at[i],
                    recv_sem=rs_recv_sems.at[i],
                    device_id=(peer(i + 1),),
                    device_id_type=pl.DeviceIdType.MESH,
                )
                rdma.start()
                rs_rdmas.append(rdma)

        for k in range(1, WORLD):
            rs_rdmas[k - 1].wait_recv()
            out_ref[rows(my), :] = out_ref[rows(my), :] + rs_buf[k - 1]

        for k in range(1, WORLD):
            rs_rdmas[k - 1].wait_send()

        ag_rdmas = []
        for k in range(1, WORLD):
            rdma = pltpu.make_async_remote_copy(
                src_ref=out_ref.at[rows(my), :],
                dst_ref=ag_buf.at[k - 1],
                send_sem=ag_send_sems.at[k - 1],
                recv_sem=ag_recv_sems.at[k - 1],
                device_id=(peer(k),),
                device_id_type=pl.DeviceIdType.MESH,
            )
            rdma.start()
            ag_rdmas.append(rdma)

        for k in range(1, WORLD):
            ag_rdmas[k - 1].wait_recv()
            src = lax.rem(my - k + WORLD, WORLD)
            out_ref[rows(src), :] = ag_buf[k - 1]

        for k in range(1, WORLD):
            ag_rdmas[k - 1].wait_send()

    return pl.pallas_call(
        body,
        out_shape=jax.ShapeDtypeStruct((N_TOK, D_OUT), jnp.bfloat16),
        in_specs=[
            pl.BlockSpec(memory_space=pltpu.VMEM),
            pl.BlockSpec(memory_space=pltpu.VMEM),
            pl.BlockSpec(memory_space=pltpu.VMEM),
        ],
        out_specs=pl.BlockSpec(memory_space=pltpu.VMEM),
        scratch_shapes=[
            pltpu.VMEM((N_PEER, CHUNK, D_OUT), jnp.bfloat16),
            pltpu.VMEM((N_PEER, CHUNK, D_OUT), jnp.bfloat16),
            pltpu.SemaphoreType.DMA((N_PEER,)),
            pltpu.SemaphoreType.DMA((N_PEER,)),
            pltpu.SemaphoreType.DMA((N_PEER,)),
            pltpu.SemaphoreType.DMA((N_PEER,)),
        ],
        compiler_params=pltpu.CompilerParams(collective_id=0),
    )(xb, route_idx, wb)


# device time: 111829 ns/iter; 1.4490x vs baseline; 1.0047x over previous
import jax
import jax.numpy as jnp
from jax import lax
from jax.experimental import pallas as pl
from jax.experimental.pallas import tpu as pltpu

WORLD = 16
N_TOK = 2048
D_IN = 512
D_OUT = 1024
E_PER_DEV = 4
CHUNK = N_TOK // WORLD
N_PEER = WORLD - 1


def kernel(x, router_W, route_idx, expert_W):
    del router_W

    xb = x.astype(jnp.bfloat16)
    wb = expert_W.astype(jnp.bfloat16)

    def body(x_ref, idx_ref, w_ref, out_ref, rs_buf,
             rs_send_sems, rs_recv_sems, ag_send_sems, ag_recv_sems):
        my = lax.axis_index("i")

        def peer(k):
            return lax.rem(my + k, WORLD)

        def rows(c):
            return pl.ds(c * CHUNK, CHUNK)

        barrier = pltpu.get_barrier_semaphore()
        for k in range(1, WORLD):
            pl.semaphore_signal(
                barrier, inc=1, device_id=(peer(k),),
                device_id_type=pl.DeviceIdType.MESH,
            )
        pl.semaphore_wait(barrier, N_PEER)

        rs_rdmas = []
        for i in range(WORLD):
            c = lax.rem(my + 1 + i, WORLD)
            xv = x_ref[rows(c), :]
            idx = idx_ref[rows(c), :]
            acc = jnp.zeros((CHUNK, D_OUT), jnp.float32)
            for e in range(E_PER_DEV):
                gid = my * E_PER_DEV + e
                xm = jnp.where(idx == gid, xv, jnp.zeros_like(xv))
                acc = acc + lax.dot(xm, w_ref[e], preferred_element_type=jnp.float32)
            out_ref[rows(c), :] = acc.astype(jnp.bfloat16)
            if i < N_PEER:
                rdma = pltpu.make_async_remote_copy(
                    src_ref=out_ref.at[rows(c), :],
                    dst_ref=rs_buf.at[i],
                    send_sem=rs_send_sems.at[i],
                    recv_sem=rs_recv_sems.at[i],
                    device_id=(peer(i + 1),),
                    device_id_type=pl.DeviceIdType.MESH,
                )
                rdma.start()
                rs_rdmas.append(rdma)

        for r in rs_rdmas:
            r.wait_recv()
        out_ref[rows(my), :] = out_ref[rows(my), :] + jnp.sum(
            rs_buf[...], axis=0, dtype=jnp.float32
        ).astype(jnp.bfloat16)

        ag_rdmas = []
        for k in range(1, WORLD):
            rdma = pltpu.make_async_remote_copy(
                src_ref=out_ref.at[rows(my), :],
                dst_ref=out_ref.at[rows(my), :],
                send_sem=ag_send_sems.at[k - 1],
                recv_sem=ag_recv_sems.at[k - 1],
                device_id=(peer(k),),
                device_id_type=pl.DeviceIdType.MESH,
            )
            rdma.start()
            ag_rdmas.append(rdma)

        for r in rs_rdmas:
            r.wait_send()
        for r in ag_rdmas:
            r.wait_recv()
        for r in ag_rdmas:
            r.wait_send()

    return pl.pallas_call(
        body,
        out_shape=jax.ShapeDtypeStruct((N_TOK, D_OUT), jnp.bfloat16),
        in_specs=[
            pl.BlockSpec(memory_space=pltpu.VMEM),
            pl.BlockSpec(memory_space=pltpu.VMEM),
            pl.BlockSpec(memory_space=pltpu.VMEM),
        ],
        out_specs=pl.BlockSpec(memory_space=pltpu.VMEM),
        scratch_shapes=[
            pltpu.VMEM((N_PEER, CHUNK, D_OUT), jnp.bfloat16),
            pltpu.SemaphoreType.DMA((N_PEER,)),
            pltpu.SemaphoreType.DMA((N_PEER,)),
            pltpu.SemaphoreType.DMA((N_PEER,)),
            pltpu.SemaphoreType.DMA((N_PEER,)),
        ],
        compiler_params=pltpu.CompilerParams(collective_id=0),
    )(xb, route_idx, wb)


# device time: 87186 ns/iter; 1.8586x vs baseline; 1.2826x over previous
import jax
import jax.numpy as jnp
from jax import lax
from jax.experimental import pallas as pl
from jax.experimental.pallas import tpu as pltpu

WORLD = 16
N_TOK = 2048
D_IN = 512
D_OUT = 1024
E_PER_DEV = 4
CAP = 192
N_PEER = WORLD - 1


def kernel(x, router_W, route_idx, expert_W):
    del router_W

    wb = expert_W.astype(jnp.bfloat16)

    route_f = route_idx.astype(jnp.float32)
    dev_T = (route_idx[:, 0] // E_PER_DEV)[None, :].astype(jnp.float32)

    def body(x_ref, route_ref, w_ref, dev_ref, out_ref,
             pt_ref, y_ref, in_buf, send_sems, recv_sems):
        my = lax.axis_index("i")

        def peer(k):
            return lax.rem(my + k, WORLD)

        barrier = pltpu.get_barrier_semaphore()
        for k in range(1, WORLD):
            pl.semaphore_signal(
                barrier, inc=1, device_id=(peer(k),),
                device_id_type=pl.DeviceIdType.MESH,
            )
        pl.semaphore_wait(barrier, N_PEER)

        iota16 = lax.broadcasted_iota(jnp.int32, (1, WORLD), 1).astype(jnp.float32)
        iota_cap = lax.broadcasted_iota(jnp.int32, (CAP, N_TOK), 0).astype(
            jnp.float32
        )

        iota16c = lax.broadcasted_iota(jnp.int32, (WORLD, 1), 0).astype(
            jnp.float32
        )
        dmask = iota16c - dev_ref[...]
        masks_T = jnp.maximum(0.0, 1.0 - dmask * dmask)

        it_r = lax.broadcasted_iota(jnp.int32, (N_TOK, N_TOK), 0)
        it_c = lax.broadcasted_iota(jnp.int32, (N_TOK, N_TOK), 1)
        tri = jnp.clip(it_c - it_r, 0, 1).astype(jnp.bfloat16)
        ranks_T = lax.dot(
            masks_T.astype(jnp.bfloat16), tri, preferred_element_type=jnp.float32
        )
        keys_T = ranks_T * masks_T - (1.0 - masks_T)

        def section_of(d):
            dd = iota16 - d.astype(jnp.float32)
            e_row = jnp.maximum(0.0, 1.0 - dd * dd)
            key_row = lax.dot(e_row, keys_T)
            dk = iota_cap - key_row
            return jnp.maximum(0.0, 1.0 - dk * dk).astype(jnp.bfloat16)

        sec_my = section_of(my)
        dims = (((0,), (0,)), ((), ()))
        sec_my_f = sec_my.astype(jnp.float32)
        xc = lax.dot(sec_my_f, x_ref[...],
                     preferred_element_type=jnp.float32).astype(jnp.bfloat16)
        routec = lax.dot(sec_my_f, route_ref[...],
                         preferred_element_type=jnp.float32)

        acc = jnp.zeros((CAP, D_OUT), jnp.float32)
        for e in range(E_PER_DEV):
            gid = (my * E_PER_DEV + e).astype(jnp.float32)
            dg = routec - gid
            m = jnp.maximum(0.0, 1.0 - dg * dg).astype(jnp.bfloat16)
            xm = xc * m
            acc = acc + lax.dot(xm, w_ref[e], preferred_element_type=jnp.float32)
        y_ref[...] = acc.astype(jnp.bfloat16)

        rdmas = []
        for k in range(1, WORLD):
            rdma = pltpu.make_async_remote_copy(
                src_ref=y_ref,
                dst_ref=in_buf.at[k - 1],
                send_sem=send_sems.at[k - 1],
                recv_sem=recv_sems.at[k - 1],
                device_id=(peer(k),),
                device_id_type=pl.DeviceIdType.MESH,
            )
            rdma.start()
            rdmas.append(rdma)

        out_ref[...] = lax.dot_general(
            sec_my, y_ref[...], dims, preferred_element_type=jnp.float32
        ).astype(jnp.bfloat16)

        for k in range(1, WORLD):
            src = lax.rem(my - k + WORLD, WORLD)
            pt_ref[pl.ds((k - 1) * CAP, CAP), :] = section_of(src)

        groups = [(0, 4), (4, 4), (8, 4), (12, 3)]
        for lo, g in groups:
            for k in range(lo + 1, lo + g + 1):
                rdmas[k - 1].wait_recv()
            p_grp = pt_ref[pl.ds(lo * CAP, g * CAP), :]
            blk = in_buf[lo:lo + g].reshape(g * CAP, D_OUT)
            out_ref[...] = out_ref[...] + lax.dot_general(
                p_grp, blk, dims, preferred_element_type=jnp.float32
            ).astype(jnp.bfloat16)

        for r in rdmas:
            r.wait_send()

    return pl.pallas_call(
        body,
        out_shape=jax.ShapeDtypeStruct((N_TOK, D_OUT), jnp.bfloat16),
        in_specs=[pl.BlockSpec(memory_space=pltpu.VMEM)] * 4,
        out_specs=pl.BlockSpec(memory_space=pltpu.VMEM),
        scratch_shapes=[
            pltpu.VMEM((N_PEER * CAP, N_TOK), jnp.bfloat16),
            pltpu.VMEM((CAP, D_OUT), jnp.bfloat16),
            pltpu.VMEM((N_PEER, CAP, D_OUT), jnp.bfloat16),
            pltpu.SemaphoreType.DMA((N_PEER,)),
            pltpu.SemaphoreType.DMA((N_PEER,)),
        ],
        compiler_params=pltpu.CompilerParams(collective_id=0),
    )(x, route_f, wb, dev_T)
